# baseline (speedup 1.0000x reference)
import jax
import jax.numpy as jnp
from jax import lax
from jax.experimental import pallas as pl
from jax.experimental.pallas import tpu as pltpu
from jax.experimental.pallas import tpu_sc as plsc

N = 10000
N2 = 10240
EG = 160000
EF = 320000
EF2 = 327680
H = 8
D = 16
ED = 16
IN_DIM = 128
HD = H * D

NC = 2
NS = 16
NW = NC * NS
EB = 128
NBLK = EF2 // EB
BLK_W = NBLK // NW
BLK_S = NBLK // NS
ROWS_PER_SUB = N2 // NS
HH = HD // NC
EPAD = EF2 - EF


def _pack_slab(x):
    xb = x.astype(jnp.bfloat16)
    x3 = xb.reshape(x.shape[0], H, D)
    lo = lax.bitcast_convert_type(x3[:, :, 0:8], jnp.uint16).astype(jnp.uint32)
    hi = lax.bitcast_convert_type(x3[:, :, 8:16], jnp.uint16).astype(jnp.uint32)
    packed = lax.bitwise_or(lo, lax.shift_left(hi, jnp.uint32(16)))
    return lax.bitcast_convert_type(packed, jnp.int32).reshape(x.shape[0], HD // 2)


def _qkv_body(h_ref, qw_ref, qb_ref, kw_ref, kb_ref, vw_ref, vb_ref,
              q_ref, k_ref, v_ref):
    hb = h_ref[...]
    q = jnp.dot(hb, qw_ref[...], preferred_element_type=jnp.float32) + qb_ref[...]
    k = jnp.dot(hb, kw_ref[...], preferred_element_type=jnp.float32) + kb_ref[...]
    v = jnp.dot(hb, vw_ref[...], preferred_element_type=jnp.float32) + vb_ref[...]
    q_ref[...] = _pack_slab(q)
    k_ref[...] = _pack_slab(k)
    v_ref[...] = _pack_slab(v)


def _qkv_proj(h_pad, Qw, Qb, Kw, Kb, Vw, Vb):
    bn = 1024
    grid = (N2 // bn,)
    row_spec = pl.BlockSpec((bn, IN_DIM), lambda i: (i, 0))
    w_spec = pl.BlockSpec((IN_DIM, HD), lambda i: (0, 0))
    b_spec = pl.BlockSpec((1, HD), lambda i: (0, 0))
    po_spec = pl.BlockSpec((bn, HD // 2), lambda i: (i, 0))
    out = jax.ShapeDtypeStruct((N2, HD // 2), jnp.int32)
    return pl.pallas_call(
        _qkv_body,
        grid=grid,
        in_specs=[row_spec, w_spec, b_spec, w_spec, b_spec, w_spec, b_spec],
        out_specs=[po_spec, po_spec, po_spec],
        out_shape=[out, out, out],
    )(h_pad, Qw, Qb.reshape(1, HD), Kw, Kb.reshape(1, HD), Vw, Vb.reshape(1, HD))


EBN = 1280


def _edata_body(rel_ref, adj_ref, e_ref, pw_ref, pb_ref, o_ref):
    i = pl.program_id(0)
    live = (i < EF // EBN).astype(jnp.float32)
    rel = rel_ref[...] * live
    adj = adj_ref[...] * live
    pe = jnp.dot(e_ref[...], pw_ref[...], preferred_element_type=jnp.float32) + pb_ref[...]
    pe = pe * (i < EG // EBN).astype(jnp.float32)
    z7 = jnp.zeros((EBN, 7), jnp.float32)
    z8 = jnp.zeros((EBN, 8), jnp.float32)
    o_ref[...] = jnp.concatenate([rel, adj, z7, pe, z8], axis=1)


def _edata_build(rel_pos, adj2, e, pe_w, pe_b):
    grid = (EF2 // EBN,)
    return pl.pallas_call(
        _edata_body,
        grid=grid,
        in_specs=[pl.BlockSpec((EBN, H), lambda i: (jnp.minimum(i, EF // EBN - 1), 0)),
                  pl.BlockSpec((EBN, 1), lambda i: (jnp.minimum(i, EF // EBN - 1), 0)),
                  pl.BlockSpec((EBN, ED), lambda i: (jnp.minimum(i, EG // EBN - 1), 0)),
                  pl.BlockSpec((ED, H), lambda i: (0, 0)),
                  pl.BlockSpec((1, H), lambda i: (0, 0))],
        out_specs=pl.BlockSpec((EBN, 32), lambda i: (i, 0)),
        out_shape=jax.ShapeDtypeStruct((EF2, 32), jnp.float32),
    )(rel_pos, adj2, e, pe_w, pe_b.reshape(1, H))


def _hout_body(wva_ref, wvb_ref, da_ref, db_ref, ex_ref, w_ref, b_ref, o_ref):
    dena = jnp.sum(da_ref[...], axis=0)
    denb = jnp.sum(db_ref[...], axis=0)
    denxa = jnp.dot(dena, ex_ref[...], preferred_element_type=jnp.float32)
    denxb = jnp.dot(denb, ex_ref[...], preferred_element_type=jnp.float32)
    wva = wva_ref[...] / jnp.maximum(denxa, 1e-30)
    wvb = wvb_ref[...] / jnp.maximum(denxb, 1e-30)
    wv = jnp.concatenate([wva, wvb], axis=1)
    o_ref[...] = jnp.dot(wv, w_ref[...], preferred_element_type=jnp.float32) + b_ref[...]


def _hout(wva, wvb, dena, denb, expand4, out_w, out_b):
    bn = 1024
    grid = (N2 // bn,)
    row_spec = pl.BlockSpec((bn, HD), lambda i: (i, 0))
    half_spec = pl.BlockSpec((bn, HH), lambda i: (i, 0))
    den_spec = pl.BlockSpec((NS, bn, 4), lambda i: (0, i, 0))
    return pl.pallas_call(
        _hout_body,
        grid=grid,
        in_specs=[half_spec, half_spec, den_spec, den_spec,
                  pl.BlockSpec((4, HH), lambda i: (0, 0)),
                  pl.BlockSpec((HD, HD), lambda i: (0, 0)),
                  pl.BlockSpec((1, HD), lambda i: (0, 0))],
        out_specs=row_spec,
        out_shape=jax.ShapeDtypeStruct((N2, HD), jnp.float32),
    )(wva, wvb, dena, denb, expand4, out_w, out_b.reshape(1, HD))


def _eout_body(s_ref, e_ref, apw_ref, apb_ref, ow_ref, ob_ref, o_ref):
    sraw = s_ref[...][:, 8:16]
    t = jnp.dot(sraw, apw_ref[...], preferred_element_type=jnp.float32)
    t = t + apb_ref[...] + e_ref[...]
    o_ref[...] = jnp.dot(t, ow_ref[...], preferred_element_type=jnp.float32) + ob_ref[...]


def _eout(exsraw, e, ap_w, ap_b, oute_w, oute_b):
    bn = 2000
    grid = (EG // bn,)
    return pl.pallas_call(
        _eout_body,
        grid=grid,
        in_specs=[pl.BlockSpec((bn, 16), lambda i: (i, 0)),
                  pl.BlockSpec((bn, ED), lambda i: (i, 0)),
                  pl.BlockSpec((H, ED), lambda i: (0, 0)),
                  pl.BlockSpec((1, ED), lambda i: (0, 0)),
                  pl.BlockSpec((ED, ED), lambda i: (0, 0)),
                  pl.BlockSpec((1, ED), lambda i: (0, 0))],
        out_specs=pl.BlockSpec((bn, ED), lambda i: (i, 0)),
        out_shape=jax.ShapeDtypeStruct((EG, ED), jnp.float32),
    )(exsraw, e, ap_w, ap_b.reshape(1, ED), oute_w, oute_b.reshape(1, ED))


def _score_kernel_body(kh_hbm, qh_hbm, ei_hbm, edata_hbm,
                       exsraw_hbm,
                       sd_v, krows, qrows, edata_v, exsraw_v,
                       sem_e, sem_g, sem_o):
    c = lax.axis_index("c")
    s = lax.axis_index("s")
    wid = s * NC + c
    start = wid * BLK_W
    lane = jnp.arange(16, dtype=jnp.int32)

    def issue_edge(b, j):
        blk = start + b
        gbase = blk * EB
        ebase = jnp.minimum(blk, EF // EB - 1) * EB
        pltpu.async_copy(ei_hbm.at[:, pl.ds(ebase, EB)], sd_v.at[j], sem_e.at[j])
        pltpu.async_copy(edata_hbm.at[pl.ds(gbase, EB), :], edata_v.at[j], sem_e.at[j])

    def wait_edge(b, j):
        blk = start + b
        gbase = blk * EB
        ebase = jnp.minimum(blk, EF // EB - 1) * EB
        pltpu.make_async_copy(ei_hbm.at[:, pl.ds(ebase, EB)], sd_v.at[j], sem_e.at[j]).wait()
        pltpu.make_async_copy(edata_hbm.at[pl.ds(gbase, EB), :], edata_v.at[j], sem_e.at[j]).wait()

    def issue_gath(j):
        pltpu.async_copy(kh_hbm.at[sd_v.at[j, 0]], krows.at[j], sem_g.at[j])
        pltpu.async_copy(qh_hbm.at[sd_v.at[j, 1]], qrows.at[j], sem_g.at[j])

    def wait_gath(j):
        pltpu.make_async_copy(kh_hbm.at[sd_v.at[j, 0]], krows.at[j], sem_g.at[j]).wait()
        pltpu.make_async_copy(qh_hbm.at[sd_v.at[j, 1]], qrows.at[j], sem_g.at[j]).wait()

    def issue_out(b, j):
        gbase = (start + b) * EB
        pltpu.async_copy(exsraw_v.at[j], exsraw_hbm.at[pl.ds(gbase, EB), :], sem_o.at[j])

    def wait_out(b, j):
        gbase = (start + b) * EB
        pltpu.make_async_copy(exsraw_v.at[j], exsraw_hbm.at[pl.ds(gbase, EB), :], sem_o.at[j]).wait()

    def compute(j):
        himask = jnp.int32(-65536)

        def group_body(g, carry2):
            eidx = g * 16 + lane
            adj2v = plsc.load_gather(edata_v.at[j], [eidx, jnp.full((16,), 8, jnp.int32)])
            for h in range(H):
                hcol = jnp.full((16,), h, jnp.int32)
                rel = plsc.load_gather(edata_v.at[j], [eidx, hcol])
                scale = adj2v * 0.25
                dot = jnp.zeros((16,), jnp.float32)
                for d2 in range(D // 2):
                    col = jnp.full((16,), h * (D // 2) + d2, jnp.int32)
                    kp = plsc.load_gather(krows.at[j], [eidx, col])
                    qp = plsc.load_gather(qrows.at[j], [eidx, col])
                    ka = plsc.bitcast(lax.shift_left(kp, 16), jnp.float32)
                    kb = plsc.bitcast(lax.bitwise_and(kp, himask), jnp.float32)
                    qa = plsc.bitcast(lax.shift_left(qp, 16), jnp.float32)
                    qb = plsc.bitcast(lax.bitwise_and(qp, himask), jnp.float32)
                    dot = dot + ka * qa + kb * qb
                acc = rel + dot * scale
                plsc.store_scatter(exsraw_v.at[j], [eidx, jnp.full((16,), 8 + h, jnp.int32)], acc)
                pe = plsc.load_gather(edata_v.at[j], [eidx, jnp.full((16,), 16 + h, jnp.int32)])
                logit = jnp.clip(acc + pe, -5.0, 5.0)
                exv = jnp.exp(logit)
                plsc.store_scatter(exsraw_v.at[j], [eidx, hcol], exv)
            return carry2
        lax.fori_loop(0, EB // 16, group_body, 0)

    issue_edge(0, 0)
    issue_edge(1, 1)
    wait_edge(0, 0)
    issue_gath(0)

    def pair_body(p, carry):
        for jj in range(2):
            b = 2 * p + jj
            nxt = 1 - jj

            @pl.when(b + 1 < BLK_W)
            def _():
                wait_edge(b + 1, nxt)
                issue_gath(nxt)

            wait_gath(jj)

            @pl.when(p >= 1)
            def _():
                wait_out(b - 2, jj)

            compute(jj)
            issue_out(b, jj)

            @pl.when(b + 2 < BLK_W)
            def _():
                issue_edge(b + 2, jj)
        return carry

    lax.fori_loop(0, BLK_W // 2, pair_body, 0)
    wait_out(BLK_W - 2, 0)
    wait_out(BLK_W - 1, 1)


def _score_pass(kh, qh, ei, edata):
    mesh = plsc.VectorSubcoreMesh(core_axis_name="c", subcore_axis_name="s",
                                  num_cores=NC, num_subcores=NS)
    return pl.kernel(
        _score_kernel_body,
        out_type=jax.ShapeDtypeStruct((EF2, 16), jnp.float32),
        mesh=mesh,
        compiler_params=pltpu.CompilerParams(needs_layout_passes=False,
                                             use_tc_tiling_on_sc=False),
        scratch_types=[
            pltpu.VMEM((2, 2, EB), jnp.int32),
            pltpu.VMEM((2, EB, HD // 2), jnp.int32),
            pltpu.VMEM((2, EB, HD // 2), jnp.int32),
            pltpu.VMEM((2, EB, 32), jnp.float32),
            pltpu.VMEM((2, EB, 16), jnp.float32),
            pltpu.SemaphoreType.DMA((2,)),
            pltpu.SemaphoreType.DMA((2,)),
            pltpu.SemaphoreType.DMA((2,)),
        ],
    )(kh, qh, ei, edata)


def _agg_kernel_body(vh2_hbm, ei_hbm, exsraw_hbm, zer_hbm,
                     wv_hbm, den_hbm,
                     sd_v, idx_v, vrows, exs_v, orows, den_t, wv_sp,
                     sem_e, sem_g):
    c = lax.axis_index("c")
    s = lax.axis_index("s")
    start = s * BLK_S
    lane = jnp.arange(16, dtype=jnp.int32)
    zeros16 = jnp.zeros((16,), jnp.float32)
    qmask = lane < 4

    pltpu.sync_copy(zer_hbm.at[pl.ds(s * ROWS_PER_SUB, ROWS_PER_SUB), :],
                    wv_sp.at[pl.ds(s * ROWS_PER_SUB, ROWS_PER_SUB), :])

    def zero_body(i, carry):
        den_t[pl.ds(i * 16, 16)] = zeros16
        return carry
    lax.fori_loop(0, (N2 * 4) // 16, zero_body, 0)
    plsc.subcore_barrier()

    def issue_edge(b, j):
        blk = start + b
        gbase = blk * EB
        ebase = jnp.minimum(blk, EF // EB - 1) * EB
        pltpu.async_copy(ei_hbm.at[:, pl.ds(ebase, EB)], sd_v.at[j], sem_e.at[j])
        pltpu.async_copy(exsraw_hbm.at[pl.ds(gbase, EB), :], exs_v.at[j], sem_e.at[j])

    def wait_edge(b, j):
        blk = start + b
        gbase = blk * EB
        ebase = jnp.minimum(blk, EF // EB - 1) * EB
        pltpu.make_async_copy(ei_hbm.at[:, pl.ds(ebase, EB)], sd_v.at[j], sem_e.at[j]).wait()
        pltpu.make_async_copy(exsraw_hbm.at[pl.ds(gbase, EB), :], exs_v.at[j], sem_e.at[j]).wait()

    def issue_gath(j):
        for k in range(EB // 16):
            sv = sd_v[j, 0, pl.ds(k * 16, 16)]
            idx_v[pl.ds(j * EB + k * 16, 16)] = sv * 2 + c
        pltpu.async_copy(vh2_hbm.at[idx_v.at[pl.ds(j * EB, EB)]], vrows.at[j], sem_g.at[j])

    def wait_gath(j):
        pltpu.make_async_copy(vh2_hbm.at[idx_v.at[pl.ds(j * EB, EB)]], vrows.at[j], sem_g.at[j]).wait()

    def compute(j):
        himask = jnp.int32(-65536)
        hi = jnp.where(lane >= 8, 1, 0).astype(jnp.int32)
        posA = hi * D + lax.bitwise_and(lane, jnp.int32(7))

        def edge_body(e, carry2):
            erow = jnp.full((16,), e, jnp.int32)
            ex01 = plsc.load_gather(exs_v.at[j], [erow, c * 4 + hi])
            ex23 = plsc.load_gather(exs_v.at[j], [erow, c * 4 + 2 + hi])
            vlo = vrows[j, e, pl.ds(0, 16)]
            vhi = vrows[j, e, pl.ds(16, 16)]
            orow = orows.at[j, e]
            plsc.store_scatter(orow, [posA],
                               plsc.bitcast(lax.shift_left(vlo, 16), jnp.float32) * ex01)
            plsc.store_scatter(orow, [posA + 8],
                               plsc.bitcast(lax.bitwise_and(vlo, himask), jnp.float32) * ex01)
            plsc.store_scatter(orow, [posA + 2 * D],
                               plsc.bitcast(lax.shift_left(vhi, 16), jnp.float32) * ex23)
            plsc.store_scatter(orow, [posA + 2 * D + 8],
                               plsc.bitcast(lax.bitwise_and(vhi, himask), jnp.float32) * ex23)
            dstb = plsc.load_gather(sd_v.at[j], [jnp.full((16,), 1, jnp.int32), erow])
            exv4 = plsc.load_gather(exs_v.at[j], [erow, c * 4 + lane])
            plsc.addupdate_scatter(den_t, [dstb * 4 + lane], exv4, mask=qmask)
            return carry2
        lax.fori_loop(0, EB, edge_body, 0)

    issue_edge(0, 0)
    issue_edge(1, 1)
    wait_edge(0, 0)
    issue_gath(0)

    def pair_body(p, carry):
        for jj in range(2):
            b = 2 * p + jj
            nxt = 1 - jj

            @pl.when(b + 1 < BLK_S)
            def _():
                wait_edge(b + 1, nxt)
                issue_gath(nxt)

            wait_gath(jj)

            @pl.when(start + b < EF // EB)
            def _():
                compute(jj)
                pltpu.sync_copy(orows.at[jj], wv_sp.at[sd_v.at[jj, 1]], add=True)

            @pl.when(b + 2 < BLK_S)
            def _():
                issue_edge(b + 2, jj)
        return carry

    lax.fori_loop(0, BLK_S // 2, pair_body, 0)

    pltpu.sync_copy(den_t, den_hbm.at[c, s])
    plsc.subcore_barrier()
    pltpu.sync_copy(wv_sp.at[pl.ds(s * ROWS_PER_SUB, ROWS_PER_SUB), :],
                    wv_hbm.at[c, pl.ds(s * ROWS_PER_SUB, ROWS_PER_SUB), :])


def _agg_pass(vh2, ei, exsraw, zer64):
    mesh = plsc.VectorSubcoreMesh(core_axis_name="c", subcore_axis_name="s",
                                  num_cores=NC, num_subcores=NS)
    return pl.kernel(
        _agg_kernel_body,
        out_type=[jax.ShapeDtypeStruct((NC, N2, HH), jnp.float32),
                  jax.ShapeDtypeStruct((NC, NS, N2 * 4), jnp.float32)],
        mesh=mesh,
        compiler_params=pltpu.CompilerParams(needs_layout_passes=False,
                                             use_tc_tiling_on_sc=False),
        scratch_types=[
            pltpu.VMEM((2, 2, EB), jnp.int32),
            pltpu.VMEM((2 * EB,), jnp.int32),
            pltpu.VMEM((2, EB, HH // 2), jnp.int32),
            pltpu.VMEM((2, EB, 16), jnp.float32),
            pltpu.VMEM((2, EB, HH), jnp.float32),
            pltpu.VMEM((N2 * 4,), jnp.float32),
            pltpu.VMEM_SHARED((N2, HH), jnp.float32),
            pltpu.SemaphoreType.DMA((2,)),
            pltpu.SemaphoreType.DMA((2,)),
        ],
    )(vh2, ei, exsraw, zer64)


@jax.jit
def _run(h, e, adj2, rel_pos_3d, Qw, Qb, Kw, Kb, Vw, Vb, pe_w, pe_b,
         ap_w, ap_b, out_w, out_b, oute_w, oute_b, ei):
    h_pad = jnp.pad(h, ((0, N2 - N), (0, 0)))
    qh_p, kh_p, vh_p = _qkv_proj(h_pad, Qw, Qb, Kw, Kb, Vw, Vb)
    edata = _edata_build(rel_pos_3d, adj2, e, pe_w, pe_b)

    zer64 = jnp.zeros((N2, HH), jnp.float32)

    exsraw = _score_pass(kh_p, qh_p, ei, edata)
    vh2 = vh_p.reshape(N2 * 2, HH // 2)
    wv, den = _agg_pass(vh2, ei, exsraw, zer64)
    den = den.reshape(NC, NS, N2, 4)

    expand4 = jnp.repeat(jnp.eye(4, dtype=jnp.float32), D, axis=1)
    h_out = _hout(wv[0], wv[1], den[0], den[1], expand4, out_w, out_b)[:N]
    e_out2 = _eout(exsraw, e, ap_w, ap_b, oute_w, oute_b)
    return h_out, e_out2


def kernel(h, e, adj2, rel_pos_3d, Qw, Qb, Kw, Kb, Vw, Vb, pe_w, pe_b,
           ap_w, ap_b, out_w, out_b, oute_w, oute_b,
           edge_index_g, edge_index_full):
    ei = edge_index_full.astype(jnp.int32)
    return _run(h, e, adj2, rel_pos_3d, Qw, Qb, Kw, Kb, Vw, Vb, pe_w, pe_b,
                ap_w, ap_b, out_w, out_b, oute_w, oute_b, ei)


# speedup vs baseline: 1.1733x; 1.1733x over previous
import jax
import jax.numpy as jnp
from jax import lax
from jax.experimental import pallas as pl
from jax.experimental.pallas import tpu as pltpu
from jax.experimental.pallas import tpu_sc as plsc

N = 10000
N2 = 10240
EG = 160000
EF = 320000
EF2 = 327680
H = 8
D = 16
ED = 16
IN_DIM = 128
HD = H * D

NC = 2
NS = 16
NW = NC * NS
EB = 128
NBLK = EF2 // EB
BLK_W = NBLK // NW
BLK_S = NBLK // NS
ROWS_PER_SUB = N2 // NS
HH = HD // NC
EPAD = EF2 - EF


def _pack_slab(x):
    xb = x.astype(jnp.bfloat16)
    x3 = xb.reshape(x.shape[0], H, D)
    lo = lax.bitcast_convert_type(x3[:, :, 0:8], jnp.uint16).astype(jnp.uint32)
    hi = lax.bitcast_convert_type(x3[:, :, 8:16], jnp.uint16).astype(jnp.uint32)
    packed = lax.bitwise_or(lo, lax.shift_left(hi, jnp.uint32(16)))
    return lax.bitcast_convert_type(packed, jnp.int32).reshape(x.shape[0], HD // 2)


def _qkv_body(h_ref, qw_ref, qb_ref, kw_ref, kb_ref, vw_ref, vb_ref,
              q_ref, k_ref, v_ref):
    hb = h_ref[...]
    q = jnp.dot(hb, qw_ref[...], preferred_element_type=jnp.float32) + qb_ref[...]
    k = jnp.dot(hb, kw_ref[...], preferred_element_type=jnp.float32) + kb_ref[...]
    v = jnp.dot(hb, vw_ref[...], preferred_element_type=jnp.float32) + vb_ref[...]
    q_ref[...] = _pack_slab(q)
    k_ref[...] = _pack_slab(k)
    v_ref[...] = _pack_slab(v)


def _qkv_proj(h_pad, Qw, Qb, Kw, Kb, Vw, Vb):
    bn = 1024
    grid = (N2 // bn,)
    row_spec = pl.BlockSpec((bn, IN_DIM), lambda i: (i, 0))
    w_spec = pl.BlockSpec((IN_DIM, HD), lambda i: (0, 0))
    b_spec = pl.BlockSpec((1, HD), lambda i: (0, 0))
    po_spec = pl.BlockSpec((bn, HD // 2), lambda i: (i, 0))
    out = jax.ShapeDtypeStruct((N2, HD // 2), jnp.int32)
    return pl.pallas_call(
        _qkv_body,
        grid=grid,
        in_specs=[row_spec, w_spec, b_spec, w_spec, b_spec, w_spec, b_spec],
        out_specs=[po_spec, po_spec, po_spec],
        out_shape=[out, out, out],
    )(h_pad, Qw, Qb.reshape(1, HD), Kw, Kb.reshape(1, HD), Vw, Vb.reshape(1, HD))


def _proj_e_body(e_ref, w_ref, b_ref, o_ref):
    o_ref[...] = jnp.dot(e_ref[...], w_ref[...], preferred_element_type=jnp.float32) + b_ref[...]


def _proj_e(e, pe_w, pe_b):
    bn = 2000
    grid = (EG // bn,)
    return pl.pallas_call(
        _proj_e_body,
        grid=grid,
        in_specs=[pl.BlockSpec((bn, ED), lambda i: (i, 0)),
                  pl.BlockSpec((ED, H), lambda i: (0, 0)),
                  pl.BlockSpec((1, H), lambda i: (0, 0))],
        out_specs=pl.BlockSpec((bn, H), lambda i: (i, 0)),
        out_shape=jax.ShapeDtypeStruct((EG, H), jnp.float32),
    )(e, pe_w, pe_b.reshape(1, H))


def _hout_body(wva_ref, wvb_ref, da_ref, db_ref, ex_ref, w_ref, b_ref, o_ref):
    dena = jnp.sum(da_ref[...], axis=0)
    denb = jnp.sum(db_ref[...], axis=0)
    denxa = jnp.dot(dena, ex_ref[...], preferred_element_type=jnp.float32)
    denxb = jnp.dot(denb, ex_ref[...], preferred_element_type=jnp.float32)
    wva = wva_ref[...] / jnp.maximum(denxa, 1e-30)
    wvb = wvb_ref[...] / jnp.maximum(denxb, 1e-30)
    wv = jnp.concatenate([wva, wvb], axis=1)
    o_ref[...] = jnp.dot(wv, w_ref[...], preferred_element_type=jnp.float32) + b_ref[...]


def _hout(wva, wvb, dena, denb, expand4, out_w, out_b):
    bn = 1000
    grid = (N // bn,)
    row_spec = pl.BlockSpec((bn, HD), lambda i: (i, 0))
    half_spec = pl.BlockSpec((bn, HH), lambda i: (i, 0))
    den_spec = pl.BlockSpec((NS, bn, 4), lambda i: (0, i, 0))
    return pl.pallas_call(
        _hout_body,
        grid=grid,
        in_specs=[half_spec, half_spec, den_spec, den_spec,
                  pl.BlockSpec((4, HH), lambda i: (0, 0)),
                  pl.BlockSpec((HD, HD), lambda i: (0, 0)),
                  pl.BlockSpec((1, HD), lambda i: (0, 0))],
        out_specs=row_spec,
        out_shape=jax.ShapeDtypeStruct((N, HD), jnp.float32),
    )(wva, wvb, dena, denb, expand4, out_w, out_b.reshape(1, HD))


def _eout_body(s_ref, e_ref, apw_ref, apb_ref, ow_ref, ob_ref, o_ref):
    sraw = s_ref[...][:, 8:16]
    t = jnp.dot(sraw, apw_ref[...], preferred_element_type=jnp.float32)
    t = t + apb_ref[...] + e_ref[...]
    o_ref[...] = jnp.dot(t, ow_ref[...], preferred_element_type=jnp.float32) + ob_ref[...]


def _eout(exsraw, e, ap_w, ap_b, oute_w, oute_b):
    bn = 2000
    grid = (EG // bn,)
    return pl.pallas_call(
        _eout_body,
        grid=grid,
        in_specs=[pl.BlockSpec((bn, 16), lambda i: (i, 0)),
                  pl.BlockSpec((bn, ED), lambda i: (i, 0)),
                  pl.BlockSpec((H, ED), lambda i: (0, 0)),
                  pl.BlockSpec((1, ED), lambda i: (0, 0)),
                  pl.BlockSpec((ED, ED), lambda i: (0, 0)),
                  pl.BlockSpec((1, ED), lambda i: (0, 0))],
        out_specs=pl.BlockSpec((bn, ED), lambda i: (i, 0)),
        out_shape=jax.ShapeDtypeStruct((EG, ED), jnp.float32),
    )(exsraw, e, ap_w, ap_b.reshape(1, ED), oute_w, oute_b.reshape(1, ED))


def _score_kernel_body(kh_hbm, qh_hbm, ei_hbm, rel_hbm, adj_hbm, pe_hbm,
                       exsraw_hbm,
                       sd_v, krows, qrows, rel_v, adj_v, pe_v, exsraw_v,
                       sem_e, sem_g, sem_p, sem_o):
    c = lax.axis_index("c")
    s = lax.axis_index("s")
    wid = s * NC + c
    start = wid * BLK_W
    lane = jnp.arange(16, dtype=jnp.int32)
    zeros16 = jnp.zeros((16,), jnp.float32)

    def issue_edge(b, j):
        blk = start + b
        ebase = jnp.minimum(blk, EF // EB - 1) * EB
        pltpu.async_copy(ei_hbm.at[:, pl.ds(ebase, EB)], sd_v.at[j], sem_e.at[j])
        pltpu.async_copy(rel_hbm.at[pl.ds(ebase, EB), :], rel_v.at[j], sem_e.at[j])
        pltpu.async_copy(adj_hbm.at[pl.ds(ebase, EB)], adj_v.at[j], sem_e.at[j])

        @pl.when(blk < EG // EB)
        def _():
            pltpu.async_copy(pe_hbm.at[pl.ds(blk * EB, EB), :], pe_v.at[j], sem_p.at[j])

    def wait_edge(b, j):
        blk = start + b
        ebase = jnp.minimum(blk, EF // EB - 1) * EB
        pltpu.make_async_copy(ei_hbm.at[:, pl.ds(ebase, EB)], sd_v.at[j], sem_e.at[j]).wait()
        pltpu.make_async_copy(rel_hbm.at[pl.ds(ebase, EB), :], rel_v.at[j], sem_e.at[j]).wait()
        pltpu.make_async_copy(adj_hbm.at[pl.ds(ebase, EB)], adj_v.at[j], sem_e.at[j]).wait()

        @pl.when(blk < EG // EB)
        def _():
            pltpu.make_async_copy(pe_hbm.at[pl.ds(blk * EB, EB), :], pe_v.at[j], sem_p.at[j]).wait()

        @pl.when(blk >= EG // EB)
        def _():
            rows2 = lax.shift_right_logical(lane, 3)
            cols8 = lax.bitwise_and(lane, jnp.int32(7))

            def zb(i, carry):
                plsc.store_scatter(pe_v.at[j], [i * 2 + rows2, cols8], zeros16)
                return carry
            lax.fori_loop(0, EB // 2, zb, 0)

    def issue_gath(j):
        pltpu.async_copy(kh_hbm.at[sd_v.at[j, 0]], krows.at[j], sem_g.at[j])
        pltpu.async_copy(qh_hbm.at[sd_v.at[j, 1]], qrows.at[j], sem_g.at[j])

    def wait_gath(j):
        pltpu.make_async_copy(kh_hbm.at[sd_v.at[j, 0]], krows.at[j], sem_g.at[j]).wait()
        pltpu.make_async_copy(qh_hbm.at[sd_v.at[j, 1]], qrows.at[j], sem_g.at[j]).wait()

    def issue_out(b, j):
        gbase = (start + b) * EB
        pltpu.async_copy(exsraw_v.at[j], exsraw_hbm.at[pl.ds(gbase, EB), :], sem_o.at[j])

    def wait_out(b, j):
        gbase = (start + b) * EB
        pltpu.make_async_copy(exsraw_v.at[j], exsraw_hbm.at[pl.ds(gbase, EB), :], sem_o.at[j]).wait()

    def compute(j):
        himask = jnp.int32(-65536)

        def group_body(g, carry2):
            eidx = g * 16 + lane
            adj2v = adj_v[j, pl.ds(g * 16, 16)]
            for h in range(H):
                hcol = jnp.full((16,), h, jnp.int32)
                rel = plsc.load_gather(rel_v.at[j], [eidx, hcol])
                scale = adj2v * 0.25
                dot = jnp.zeros((16,), jnp.float32)
                for d2 in range(D // 2):
                    col = jnp.full((16,), h * (D // 2) + d2, jnp.int32)
                    kp = plsc.load_gather(krows.at[j], [eidx, col])
                    qp = plsc.load_gather(qrows.at[j], [eidx, col])
                    ka = plsc.bitcast(lax.shift_left(kp, 16), jnp.float32)
                    kb = plsc.bitcast(lax.bitwise_and(kp, himask), jnp.float32)
                    qa = plsc.bitcast(lax.shift_left(qp, 16), jnp.float32)
                    qb = plsc.bitcast(lax.bitwise_and(qp, himask), jnp.float32)
                    dot = dot + ka * qa + kb * qb
                acc = rel + dot * scale
                plsc.store_scatter(exsraw_v.at[j], [eidx, jnp.full((16,), 8 + h, jnp.int32)], acc)
                pe = plsc.load_gather(pe_v.at[j], [eidx, hcol])
                logit = jnp.clip(acc + pe, -5.0, 5.0)
                exv = jnp.exp(logit)
                plsc.store_scatter(exsraw_v.at[j], [eidx, hcol], exv)
            return carry2
        lax.fori_loop(0, EB // 16, group_body, 0)

    issue_edge(0, 0)
    issue_edge(1, 1)
    wait_edge(0, 0)
    issue_gath(0)

    def pair_body(p, carry):
        for jj in range(2):
            b = 2 * p + jj
            nxt = 1 - jj

            @pl.when(b + 1 < BLK_W)
            def _():
                wait_edge(b + 1, nxt)
                issue_gath(nxt)

            wait_gath(jj)

            @pl.when(p >= 1)
            def _():
                wait_out(b - 2, jj)

            compute(jj)
            issue_out(b, jj)

            @pl.when(b + 2 < BLK_W)
            def _():
                issue_edge(b + 2, jj)
        return carry

    lax.fori_loop(0, BLK_W // 2, pair_body, 0)
    wait_out(BLK_W - 2, 0)
    wait_out(BLK_W - 1, 1)


def _score_pass(kh, qh, ei, rel, adjf, pe):
    mesh = plsc.VectorSubcoreMesh(core_axis_name="c", subcore_axis_name="s",
                                  num_cores=NC, num_subcores=NS)
    return pl.kernel(
        _score_kernel_body,
        out_type=jax.ShapeDtypeStruct((EF2, 16), jnp.float32),
        mesh=mesh,
        compiler_params=pltpu.CompilerParams(needs_layout_passes=False,
                                             use_tc_tiling_on_sc=False),
        scratch_types=[
            pltpu.VMEM((2, 2, EB), jnp.int32),
            pltpu.VMEM((2, EB, HD // 2), jnp.int32),
            pltpu.VMEM((2, EB, HD // 2), jnp.int32),
            pltpu.VMEM((2, EB, 8), jnp.float32),
            pltpu.VMEM((2, EB), jnp.float32),
            pltpu.VMEM((2, EB, 8), jnp.float32),
            pltpu.VMEM((2, EB, 16), jnp.float32),
            pltpu.SemaphoreType.DMA((2,)),
            pltpu.SemaphoreType.DMA((2,)),
            pltpu.SemaphoreType.DMA((2,)),
            pltpu.SemaphoreType.DMA((2,)),
        ],
    )(kh, qh, ei, rel, adjf, pe)


def _agg_kernel_body(vh2_hbm, ei_hbm, exsraw_hbm, zer_hbm,
                     wv_hbm, den_hbm,
                     sd_v, idx_v, vrows, exs_v, orows, den_t, wv_sp,
                     sem_e, sem_g):
    c = lax.axis_index("c")
    s = lax.axis_index("s")
    start = s * BLK_S
    lane = jnp.arange(16, dtype=jnp.int32)
    zeros16 = jnp.zeros((16,), jnp.float32)
    qmask = lane < 4

    pltpu.sync_copy(zer_hbm.at[pl.ds(s * ROWS_PER_SUB, ROWS_PER_SUB), :],
                    wv_sp.at[pl.ds(s * ROWS_PER_SUB, ROWS_PER_SUB), :])

    def zero_body(i, carry):
        den_t[pl.ds(i * 16, 16)] = zeros16
        return carry
    lax.fori_loop(0, (N2 * 4) // 16, zero_body, 0)
    plsc.subcore_barrier()

    def issue_edge(b, j):
        blk = start + b
        gbase = blk * EB
        ebase = jnp.minimum(blk, EF // EB - 1) * EB
        pltpu.async_copy(ei_hbm.at[:, pl.ds(ebase, EB)], sd_v.at[j], sem_e.at[j])
        pltpu.async_copy(exsraw_hbm.at[pl.ds(gbase, EB), :], exs_v.at[j], sem_e.at[j])

    def wait_edge(b, j):
        blk = start + b
        gbase = blk * EB
        ebase = jnp.minimum(blk, EF // EB - 1) * EB
        pltpu.make_async_copy(ei_hbm.at[:, pl.ds(ebase, EB)], sd_v.at[j], sem_e.at[j]).wait()
        pltpu.make_async_copy(exsraw_hbm.at[pl.ds(gbase, EB), :], exs_v.at[j], sem_e.at[j]).wait()

    def issue_gath(j):
        for k in range(EB // 16):
            sv = sd_v[j, 0, pl.ds(k * 16, 16)]
            idx_v[pl.ds(j * EB + k * 16, 16)] = sv * 2 + c
        pltpu.async_copy(vh2_hbm.at[idx_v.at[pl.ds(j * EB, EB)]], vrows.at[j], sem_g.at[j])

    def wait_gath(j):
        pltpu.make_async_copy(vh2_hbm.at[idx_v.at[pl.ds(j * EB, EB)]], vrows.at[j], sem_g.at[j]).wait()

    def compute(j):
        himask = jnp.int32(-65536)
        hi = jnp.where(lane >= 8, 1, 0).astype(jnp.int32)
        posA = hi * D + lax.bitwise_and(lane, jnp.int32(7))

        def edge_body(e, carry2):
            erow = jnp.full((16,), e, jnp.int32)
            ex01 = plsc.load_gather(exs_v.at[j], [erow, c * 4 + hi])
            ex23 = plsc.load_gather(exs_v.at[j], [erow, c * 4 + 2 + hi])
            vlo = vrows[j, e, pl.ds(0, 16)]
            vhi = vrows[j, e, pl.ds(16, 16)]
            orow = orows.at[j, e]
            plsc.store_scatter(orow, [posA],
                               plsc.bitcast(lax.shift_left(vlo, 16), jnp.float32) * ex01)
            plsc.store_scatter(orow, [posA + 8],
                               plsc.bitcast(lax.bitwise_and(vlo, himask), jnp.float32) * ex01)
            plsc.store_scatter(orow, [posA + 2 * D],
                               plsc.bitcast(lax.shift_left(vhi, 16), jnp.float32) * ex23)
            plsc.store_scatter(orow, [posA + 2 * D + 8],
                               plsc.bitcast(lax.bitwise_and(vhi, himask), jnp.float32) * ex23)
            dstb = plsc.load_gather(sd_v.at[j], [jnp.full((16,), 1, jnp.int32), erow])
            exv4 = plsc.load_gather(exs_v.at[j], [erow, c * 4 + lane])
            plsc.addupdate_scatter(den_t, [dstb * 4 + lane], exv4, mask=qmask)
            return carry2
        lax.fori_loop(0, EB, edge_body, 0)

    issue_edge(0, 0)
    issue_edge(1, 1)
    wait_edge(0, 0)
    issue_gath(0)

    def pair_body(p, carry):
        for jj in range(2):
            b = 2 * p + jj
            nxt = 1 - jj

            @pl.when(b + 1 < BLK_S)
            def _():
                wait_edge(b + 1, nxt)
                issue_gath(nxt)

            wait_gath(jj)

            @pl.when(start + b < EF // EB)
            def _():
                compute(jj)
                pltpu.sync_copy(orows.at[jj], wv_sp.at[sd_v.at[jj, 1]], add=True)

            @pl.when(b + 2 < BLK_S)
            def _():
                issue_edge(b + 2, jj)
        return carry

    lax.fori_loop(0, BLK_S // 2, pair_body, 0)

    pltpu.sync_copy(den_t, den_hbm.at[c, s])
    plsc.subcore_barrier()
    pltpu.sync_copy(wv_sp.at[pl.ds(s * ROWS_PER_SUB, ROWS_PER_SUB), :],
                    wv_hbm.at[c, pl.ds(s * ROWS_PER_SUB, ROWS_PER_SUB), :])


def _agg_pass(vh2, ei, exsraw, zer64):
    mesh = plsc.VectorSubcoreMesh(core_axis_name="c", subcore_axis_name="s",
                                  num_cores=NC, num_subcores=NS)
    return pl.kernel(
        _agg_kernel_body,
        out_type=[jax.ShapeDtypeStruct((NC, N2, HH), jnp.float32),
                  jax.ShapeDtypeStruct((NC, NS, N2 * 4), jnp.float32)],
        mesh=mesh,
        compiler_params=pltpu.CompilerParams(needs_layout_passes=False,
                                             use_tc_tiling_on_sc=False),
        scratch_types=[
            pltpu.VMEM((2, 2, EB), jnp.int32),
            pltpu.VMEM((2 * EB,), jnp.int32),
            pltpu.VMEM((2, EB, HH // 2), jnp.int32),
            pltpu.VMEM((2, EB, 16), jnp.float32),
            pltpu.VMEM((2, EB, HH), jnp.float32),
            pltpu.VMEM((N2 * 4,), jnp.float32),
            pltpu.VMEM_SHARED((N2, HH), jnp.float32),
            pltpu.SemaphoreType.DMA((2,)),
            pltpu.SemaphoreType.DMA((2,)),
        ],
    )(vh2, ei, exsraw, zer64)


@jax.jit
def _run(h, e, adj2, rel_pos_3d, Qw, Qb, Kw, Kb, Vw, Vb, pe_w, pe_b,
         ap_w, ap_b, out_w, out_b, oute_w, oute_b, ei):
    h_pad = jnp.pad(h, ((0, N2 - N), (0, 0)))
    qh_p, kh_p, vh_p = _qkv_proj(h_pad, Qw, Qb, Kw, Kb, Vw, Vb)
    proje = _proj_e(e, pe_w, pe_b)

    zer64 = jnp.zeros((N2, HH), jnp.float32)

    exsraw = _score_pass(kh_p, qh_p, ei, rel_pos_3d, adj2[:, 0], proje)
    vh2 = vh_p.reshape(N2 * 2, HH // 2)
    wv, den = _agg_pass(vh2, ei, exsraw, zer64)
    den = den.reshape(NC, NS, N2, 4)

    expand4 = jnp.repeat(jnp.eye(4, dtype=jnp.float32), D, axis=1)
    h_out = _hout(wv[0], wv[1], den[0], den[1], expand4, out_w, out_b)
    e_out2 = _eout(exsraw, e, ap_w, ap_b, oute_w, oute_b)
    return h_out, e_out2


def kernel(h, e, adj2, rel_pos_3d, Qw, Qb, Kw, Kb, Vw, Vb, pe_w, pe_b,
           ap_w, ap_b, out_w, out_b, oute_w, oute_b,
           edge_index_g, edge_index_full):
    ei = edge_index_full.astype(jnp.int32)
    return _run(h, e, adj2, rel_pos_3d, Qw, Qb, Kw, Kb, Vw, Vb, pe_w, pe_b,
                ap_w, ap_b, out_w, out_b, oute_w, oute_b, ei)
